# baseline (device time: 209481 ns/iter reference)
import jax
import jax.numpy as jnp
from jax import lax
from jax.experimental import pallas as pl
from jax.experimental.pallas import tpu as pltpu

CHUNK = 1024


def kernel(x):
    m, n = x.shape
    half = n // 2
    n_chunks = m // CHUNK

    def body(x_hbm, out_hbm, in_buf, send_buf, local_buf,
             in_sems, out_sems, send_sems, recv_sems):
        my_x = lax.axis_index("x")
        my_y = lax.axis_index("y")
        my_z = lax.axis_index("z")
        peer_z = 1 - my_z
        peer = (my_x, my_y, peer_z)

        barrier_sem = pltpu.get_barrier_semaphore()
        pl.semaphore_signal(
            barrier_sem, inc=1,
            device_id=peer, device_id_type=pl.DeviceIdType.MESH,
        )
        pl.semaphore_wait(barrier_sem, 1)

        def start_load(i):
            cp = pltpu.make_async_copy(
                x_hbm.at[pl.ds(i * CHUNK, CHUNK), :],
                in_buf.at[i % 2],
                in_sems.at[i % 2],
            )
            cp.start()
            return cp

        loads = [start_load(0)]
        rdmas = []
        outs = []
        for i in range(n_chunks):
            if i + 1 < n_chunks:
                loads.append(start_load(i + 1))
            loads[i].wait()
            r0 = i * CHUNK

            send_buf[pl.ds(r0, CHUNK), :] = in_buf[
                i % 2, :, pl.ds(peer_z * half, half)
            ].astype(jnp.bfloat16)
            rdma = pltpu.make_async_remote_copy(
                src_ref=send_buf.at[pl.ds(r0, CHUNK), :],
                dst_ref=out_hbm.at[pl.ds(my_z * m + r0, CHUNK), :],
                send_sem=send_sems.at[i],
                recv_sem=recv_sems.at[i],
                device_id=peer,
                device_id_type=pl.DeviceIdType.MESH,
            )
            rdma.start()
            rdmas.append(rdma)

            local_buf[pl.ds(r0, CHUNK), :] = in_buf[
                i % 2, :, pl.ds(my_z * half, half)
            ].astype(jnp.bfloat16)
            cp_out = pltpu.make_async_copy(
                local_buf.at[pl.ds(r0, CHUNK), :],
                out_hbm.at[pl.ds(my_z * m + r0, CHUNK), :],
                out_sems.at[i],
            )
            cp_out.start()
            outs.append(cp_out)

        for cp in outs:
            cp.wait()
        for rdma in rdmas:
            rdma.wait()

    out_shape = jax.ShapeDtypeStruct((2 * m, half), jnp.bfloat16)
    return pl.pallas_call(
        body,
        out_shape=out_shape,
        in_specs=[pl.BlockSpec(memory_space=pl.ANY)],
        out_specs=pl.BlockSpec(memory_space=pl.ANY),
        scratch_shapes=[
            pltpu.VMEM((2, CHUNK, n), jnp.float32),
            pltpu.VMEM((m, half), jnp.bfloat16),
            pltpu.VMEM((m, half), jnp.bfloat16),
            pltpu.SemaphoreType.DMA((2,)),
            pltpu.SemaphoreType.DMA((n_chunks,)),
            pltpu.SemaphoreType.DMA((n_chunks,)),
            pltpu.SemaphoreType.DMA((n_chunks,)),
        ],
        compiler_params=pltpu.CompilerParams(
            collective_id=0,
            vmem_limit_bytes=56 * 1024 * 1024,
        ),
    )(x)


# device time: 208331 ns/iter; 1.0055x vs baseline; 1.0055x over previous
import jax
import jax.numpy as jnp
from jax import lax
from jax.experimental import pallas as pl
from jax.experimental.pallas import tpu as pltpu

CHUNK = 512


def kernel(x):
    m, n = x.shape
    half = n // 2
    n_chunks = m // CHUNK

    def body(x_hbm, out_hbm, in_buf, send_buf, local_buf,
             in_sems, out_sems, send_sems, recv_sems):
        my_x = lax.axis_index("x")
        my_y = lax.axis_index("y")
        my_z = lax.axis_index("z")
        peer_z = 1 - my_z
        peer = (my_x, my_y, peer_z)

        barrier_sem = pltpu.get_barrier_semaphore()
        pl.semaphore_signal(
            barrier_sem, inc=1,
            device_id=peer, device_id_type=pl.DeviceIdType.MESH,
        )

        def start_load(i):
            cp = pltpu.make_async_copy(
                x_hbm.at[pl.ds(i * CHUNK, CHUNK), :],
                in_buf.at[i % 2],
                in_sems.at[i % 2],
            )
            cp.start()
            return cp

        loads = [start_load(0)]
        rdmas = []
        outs = []
        for i in range(n_chunks):
            if i + 1 < n_chunks:
                loads.append(start_load(i + 1))
            loads[i].wait()
            r0 = i * CHUNK

            send_buf[pl.ds(r0, CHUNK), :] = in_buf[
                i % 2, :, pl.ds(peer_z * half, half)
            ].astype(jnp.bfloat16)
            if i == 0:
                pl.semaphore_wait(barrier_sem, 1)
            rdma = pltpu.make_async_remote_copy(
                src_ref=send_buf.at[pl.ds(r0, CHUNK), :],
                dst_ref=out_hbm.at[pl.ds(my_z * m + r0, CHUNK), :],
                send_sem=send_sems.at[i],
                recv_sem=recv_sems.at[i],
                device_id=peer,
                device_id_type=pl.DeviceIdType.MESH,
            )
            rdma.start()
            rdmas.append(rdma)

            local_buf[pl.ds(r0, CHUNK), :] = in_buf[
                i % 2, :, pl.ds(my_z * half, half)
            ].astype(jnp.bfloat16)
            cp_out = pltpu.make_async_copy(
                local_buf.at[pl.ds(r0, CHUNK), :],
                out_hbm.at[pl.ds(my_z * m + r0, CHUNK), :],
                out_sems.at[i],
            )
            cp_out.start()
            outs.append(cp_out)

        for cp in outs:
            cp.wait()
        for rdma in rdmas:
            rdma.wait()

    out_shape = jax.ShapeDtypeStruct((2 * m, half), jnp.bfloat16)
    return pl.pallas_call(
        body,
        out_shape=out_shape,
        in_specs=[pl.BlockSpec(memory_space=pl.ANY)],
        out_specs=pl.BlockSpec(memory_space=pl.ANY),
        scratch_shapes=[
            pltpu.VMEM((2, CHUNK, n), jnp.float32),
            pltpu.VMEM((m, half), jnp.bfloat16),
            pltpu.VMEM((m, half), jnp.bfloat16),
            pltpu.SemaphoreType.DMA((2,)),
            pltpu.SemaphoreType.DMA((n_chunks,)),
            pltpu.SemaphoreType.DMA((n_chunks,)),
            pltpu.SemaphoreType.DMA((n_chunks,)),
        ],
        compiler_params=pltpu.CompilerParams(
            collective_id=0,
            vmem_limit_bytes=56 * 1024 * 1024,
        ),
    )(x)
